# Initial kernel scaffold; baseline (speedup 1.0000x reference)
import jax
import jax.numpy as jnp
from jax.experimental import pallas as pl


def kernel(pc):
    raise NotImplementedError("write your pallas kernel here")


# speedup vs baseline: 8.4936x; 8.4936x over previous
import functools

import jax
import jax.numpy as jnp
from jax import lax
from jax.experimental import pallas as pl
from jax.experimental.pallas import tpu as pltpu
from jax.experimental.pallas import tpu_sc as plsc

_N = 4096
_K = 32
_R = 128
_BIG = 2 ** 30
_SC_CHUNK = 4096


def _topk_body(pc_ref, pcrt_ref, out_ref, dist_ref, sq_ref):
    b = pl.program_id(0)
    i = pl.program_id(1)
    pc_all = pc_ref[0]
    prt = pcrt_ref[0]

    @pl.when(i == 0)
    def _():
        sq_ref[...] = jnp.sum(pc_all * pc_all, axis=1, keepdims=True)

    sq_j = sq_ref[...]
    sq_r = jnp.sum(prt * prt, axis=0, keepdims=True)
    dot = lax.dot_general(pc_all, prt, (((1,), (0,)), ((), ())),
                          preferred_element_type=jnp.float32)
    dist_ref[...] = (sq_j + sq_r) - 2.0 * dot

    iota_c = lax.broadcasted_iota(jnp.int32, (_N, _R), 0)
    sel_iota = lax.broadcasted_iota(jnp.int32, (_K, _R), 0)

    def step(t, acc):
        d = dist_ref[...]
        m = jnp.min(d, axis=0, keepdims=True)
        iw = jnp.where(d == m, iota_c, _BIG)
        idx = jnp.min(iw, axis=0, keepdims=True)
        dist_ref[...] = jnp.where(iota_c == idx, jnp.inf, d)
        return jnp.where(sel_iota == t, jnp.broadcast_to(idx, (_K, _R)), acc)

    acc = lax.fori_loop(0, _K, step, jnp.zeros((_K, _R), jnp.int32))
    out_ref[0] = acc + b * _N


def _topk_edges(pc):
    B = pc.shape[0]
    pcT = jnp.swapaxes(pc, 1, 2)
    edges_t = pl.pallas_call(
        _topk_body,
        grid=(B, _N // _R),
        in_specs=[
            pl.BlockSpec((1, _N, 3), lambda b, i: (b, 0, 0)),
            pl.BlockSpec((1, 3, _R), lambda b, i: (b, 0, i)),
        ],
        out_specs=pl.BlockSpec((1, _K, _R), lambda b, i: (b, 0, i)),
        out_shape=jax.ShapeDtypeStruct((B, _K, _N), jnp.int32),
        scratch_shapes=[
            pltpu.VMEM((_N, _R), jnp.float32),
            pltpu.VMEM((_N, 1), jnp.float32),
        ],
    )(pc, pcT)
    return jnp.swapaxes(edges_t, 1, 2).reshape(-1)


def _edge_feats(px, py, pz, edges):
    npts = px.shape[0]
    e_total = edges.shape[0]
    info = plsc.get_sparse_core_info()
    nw = info.num_cores * info.num_subcores
    per_tile = e_total // nw
    n_chunks = per_tile // _SC_CHUNK
    mesh = plsc.VectorSubcoreMesh(core_axis_name="c", subcore_axis_name="s")

    @functools.partial(
        pl.kernel,
        mesh=mesh,
        compiler_params=pltpu.CompilerParams(needs_layout_passes=False),
        out_type=[jax.ShapeDtypeStruct((e_total,), jnp.float32)] * 3,
        scratch_types=[
            pltpu.VMEM((npts,), jnp.float32),
            pltpu.VMEM((npts,), jnp.float32),
            pltpu.VMEM((npts,), jnp.float32),
            pltpu.VMEM((_SC_CHUNK,), jnp.int32),
            pltpu.VMEM((_SC_CHUNK,), jnp.float32),
            pltpu.VMEM((_SC_CHUNK,), jnp.float32),
            pltpu.VMEM((_SC_CHUNK,), jnp.float32),
        ],
    )
    def k(px_hbm, py_hbm, pz_hbm, e_hbm, ox_hbm, oy_hbm, oz_hbm,
          pxv, pyv, pzv, idxv, bx, by, bz):
        wid = lax.axis_index("s") * info.num_cores + lax.axis_index("c")
        pltpu.sync_copy(px_hbm, pxv)
        pltpu.sync_copy(py_hbm, pyv)
        pltpu.sync_copy(pz_hbm, pzv)
        lane = lax.iota(jnp.int32, 16)

        def chunk(c, _):
            base = wid * per_tile + c * _SC_CHUNK
            pltpu.sync_copy(e_hbm.at[pl.ds(base, _SC_CHUNK)], idxv)

            def group(g, _):
                idx16 = idxv[pl.ds(g * 16, 16)]
                org16 = lax.shift_right_logical(base + g * 16 + lane, 5)
                bx[pl.ds(g * 16, 16)] = (
                    plsc.load_gather(pxv, [idx16])
                    - plsc.load_gather(pxv, [org16]))
                by[pl.ds(g * 16, 16)] = (
                    plsc.load_gather(pyv, [idx16])
                    - plsc.load_gather(pyv, [org16]))
                bz[pl.ds(g * 16, 16)] = (
                    plsc.load_gather(pzv, [idx16])
                    - plsc.load_gather(pzv, [org16]))
                return 0

            lax.fori_loop(0, _SC_CHUNK // 16, group, 0)
            pltpu.sync_copy(bx, ox_hbm.at[pl.ds(base, _SC_CHUNK)])
            pltpu.sync_copy(by, oy_hbm.at[pl.ds(base, _SC_CHUNK)])
            pltpu.sync_copy(bz, oz_hbm.at[pl.ds(base, _SC_CHUNK)])
            return 0

        lax.fori_loop(0, n_chunks, chunk, 0)

    return k(px, py, pz, edges)


def kernel(pc):
    B, n, _ = pc.shape
    edges = _topk_edges(pc)
    pcf = pc.reshape(B * n, 3)
    fx, fy, fz = _edge_feats(pcf[:, 0], pcf[:, 1], pcf[:, 2], edges)
    feats = jnp.stack([fx, fy, fz], axis=-1)
    return edges, feats


# speedup vs baseline: 22.4516x; 2.6434x over previous
import functools

import jax
import jax.numpy as jnp
import numpy as np
from jax import lax
from jax.experimental import pallas as pl
from jax.experimental.pallas import tpu as pltpu
from jax.experimental.pallas import tpu_sc as plsc
from jax.sharding import Mesh, PartitionSpec as P

_N = 4096
_K = 32
_R = 128
_SC_CHUNK = 4096


def _topk_body(pc_ref, pcrt_ref, out_ref, dist_ref, sq_ref):
    b = pl.program_id(0)
    i = pl.program_id(1)
    pc_all = pc_ref[0]
    prt = pcrt_ref[0]

    @pl.when(i == 0)
    def _():
        sq_ref[...] = jnp.sum(pc_all * pc_all, axis=1, keepdims=True)

    sq_j = sq_ref[...]
    sq_r = jnp.sum(prt * prt, axis=0, keepdims=True)
    dot = lax.dot_general(pc_all, prt, (((1,), (0,)), ((), ())),
                          preferred_element_type=jnp.float32)
    dist_ref[...] = (sq_j + sq_r) - 2.0 * dot

    iota_c = lax.broadcasted_iota(jnp.int32, (_N, _R), 0)
    sel_iota = lax.broadcasted_iota(jnp.int32, (_K, _R), 0)

    def step(t, carry):
        acc, idx_prev = carry
        d = jnp.where(iota_c == idx_prev, jnp.inf, dist_ref[...])
        dist_ref[...] = d
        m = jnp.min(d, axis=0, keepdims=True)
        iw = jnp.where(d == m, iota_c, _N)
        idx = jnp.min(iw, axis=0, keepdims=True)
        acc = jnp.where(sel_iota == t, jnp.broadcast_to(idx, (_K, _R)), acc)
        return acc, idx

    acc, _ = lax.fori_loop(
        0, _K, step,
        (jnp.zeros((_K, _R), jnp.int32), jnp.full((1, _R), -1, jnp.int32)))
    out_ref[0] = acc + b * _N


def _topk_edges(pc):
    B = pc.shape[0]
    pcT = jnp.swapaxes(pc, 1, 2)
    edges_t = pl.pallas_call(
        _topk_body,
        grid=(B, _N // _R),
        in_specs=[
            pl.BlockSpec((1, _N, 3), lambda b, i: (b, 0, 0)),
            pl.BlockSpec((1, 3, _R), lambda b, i: (b, 0, i)),
        ],
        out_specs=pl.BlockSpec((1, _K, _R), lambda b, i: (b, 0, i)),
        out_shape=jax.ShapeDtypeStruct((B, _K, _N), jnp.int32),
        scratch_shapes=[
            pltpu.VMEM((_N, _R), jnp.float32),
            pltpu.VMEM((_N, 1), jnp.float32),
        ],
    )(pc, pcT)
    return jnp.swapaxes(edges_t, 1, 2).reshape(-1)


def _edge_feats(px, py, pz, edges):
    npts = px.shape[0]
    e_total = edges.shape[0]
    info = plsc.get_sparse_core_info()
    nw = info.num_cores * info.num_subcores
    per_tile = e_total // nw
    n_chunks = per_tile // _SC_CHUNK
    mesh = plsc.VectorSubcoreMesh(core_axis_name="c", subcore_axis_name="s")

    @functools.partial(
        pl.kernel,
        mesh=mesh,
        compiler_params=pltpu.CompilerParams(needs_layout_passes=False),
        out_type=[jax.ShapeDtypeStruct((e_total,), jnp.float32)] * 3,
        scratch_types=[
            pltpu.VMEM((npts,), jnp.float32),
            pltpu.VMEM((npts,), jnp.float32),
            pltpu.VMEM((npts,), jnp.float32),
            pltpu.VMEM((_SC_CHUNK,), jnp.int32),
            pltpu.VMEM((_SC_CHUNK,), jnp.float32),
            pltpu.VMEM((_SC_CHUNK,), jnp.float32),
            pltpu.VMEM((_SC_CHUNK,), jnp.float32),
        ],
    )
    def k(px_hbm, py_hbm, pz_hbm, e_hbm, ox_hbm, oy_hbm, oz_hbm,
          pxv, pyv, pzv, idxv, bx, by, bz):
        wid = lax.axis_index("s") * info.num_cores + lax.axis_index("c")
        pltpu.sync_copy(px_hbm, pxv)
        pltpu.sync_copy(py_hbm, pyv)
        pltpu.sync_copy(pz_hbm, pzv)
        lane = lax.iota(jnp.int32, 16)

        def chunk(c, _):
            base = wid * per_tile + c * _SC_CHUNK
            pltpu.sync_copy(e_hbm.at[pl.ds(base, _SC_CHUNK)], idxv)

            def group(g, _):
                idx16 = idxv[pl.ds(g * 16, 16)]
                org16 = lax.shift_right_logical(base + g * 16 + lane, 5)
                bx[pl.ds(g * 16, 16)] = (
                    plsc.load_gather(pxv, [idx16])
                    - plsc.load_gather(pxv, [org16]))
                by[pl.ds(g * 16, 16)] = (
                    plsc.load_gather(pyv, [idx16])
                    - plsc.load_gather(pyv, [org16]))
                bz[pl.ds(g * 16, 16)] = (
                    plsc.load_gather(pzv, [idx16])
                    - plsc.load_gather(pzv, [org16]))
                return 0

            lax.fori_loop(0, _SC_CHUNK // 16, group, 0)
            pltpu.sync_copy(bx, ox_hbm.at[pl.ds(base, _SC_CHUNK)])
            pltpu.sync_copy(by, oy_hbm.at[pl.ds(base, _SC_CHUNK)])
            pltpu.sync_copy(bz, oz_hbm.at[pl.ds(base, _SC_CHUNK)])
            return 0

        lax.fori_loop(0, n_chunks, chunk, 0)

    return k(px, py, pz, edges)


def _shard_fn(pc_local):
    bl, n, _ = pc_local.shape
    edges_local = _topk_edges(pc_local)
    pcf = pc_local.reshape(bl * n, 3)
    fx, fy, fz = _edge_feats(pcf[:, 0], pcf[:, 1], pcf[:, 2], edges_local)
    feats = jnp.stack([fx, fy, fz], axis=-1)
    edges = edges_local + lax.axis_index("d") * (bl * n)
    return edges, feats


def kernel(pc):
    nd = 2 if len(jax.devices()) >= 2 else 1
    mesh = Mesh(np.array(jax.devices()[:nd]), ("d",))
    edges, feats = jax.shard_map(
        _shard_fn, mesh=mesh, in_specs=P("d"), out_specs=(P("d"), P("d")),
        check_vma=False,
    )(pc)
    return edges, feats


# speedup vs baseline: 24.8370x; 1.1062x over previous
import functools

import jax
import jax.numpy as jnp
import numpy as np
from jax import lax
from jax.experimental import pallas as pl
from jax.experimental.pallas import tpu as pltpu
from jax.experimental.pallas import tpu_sc as plsc
from jax.sharding import Mesh, PartitionSpec as P

_N = 4096
_K = 32
_R = 256
_SC_CHUNK = 4096


def _topk_body(pc_ref, pcrt_ref, out_ref, dist_ref, sq_ref):
    b = pl.program_id(0)
    i = pl.program_id(1)
    pc_all = pc_ref[0]
    prt = pcrt_ref[0]

    @pl.when(i == 0)
    def _():
        sq_ref[...] = jnp.sum(pc_all * pc_all, axis=1, keepdims=True)

    sq_j = sq_ref[...]
    sq_r = jnp.sum(prt * prt, axis=0, keepdims=True)
    dot = lax.dot_general(pc_all, prt, (((1,), (0,)), ((), ())),
                          preferred_element_type=jnp.float32)
    dist_ref[...] = (sq_j + sq_r) - 2.0 * dot

    iota_c = lax.broadcasted_iota(jnp.int32, (_N, _R), 0)
    sel_iota = lax.broadcasted_iota(jnp.int32, (_K, _R), 0)

    def step(t, carry):
        acc, idx_prev = carry
        d = jnp.where(iota_c == idx_prev, jnp.inf, dist_ref[...])
        dist_ref[...] = d
        m = jnp.min(d, axis=0, keepdims=True)
        iw = jnp.where(d == m, iota_c, _N)
        idx = jnp.min(iw, axis=0, keepdims=True)
        acc = jnp.where(sel_iota == t, jnp.broadcast_to(idx, (_K, _R)), acc)
        return acc, idx

    acc, _ = lax.fori_loop(
        0, _K, step,
        (jnp.zeros((_K, _R), jnp.int32), jnp.full((1, _R), -1, jnp.int32)))
    out_ref[0] = acc + b * _N


def _topk_edges(pc):
    B = pc.shape[0]
    pcT = jnp.swapaxes(pc, 1, 2)
    edges_t = pl.pallas_call(
        _topk_body,
        grid=(B, _N // _R),
        in_specs=[
            pl.BlockSpec((1, _N, 3), lambda b, i: (b, 0, 0)),
            pl.BlockSpec((1, 3, _R), lambda b, i: (b, 0, i)),
        ],
        out_specs=pl.BlockSpec((1, _K, _R), lambda b, i: (b, 0, i)),
        out_shape=jax.ShapeDtypeStruct((B, _K, _N), jnp.int32),
        scratch_shapes=[
            pltpu.VMEM((_N, _R), jnp.float32),
            pltpu.VMEM((_N, 1), jnp.float32),
        ],
    )(pc, pcT)
    return jnp.swapaxes(edges_t, 1, 2).reshape(-1)


def _edge_feats(px, py, pz, edges):
    npts = px.shape[0]
    e_total = edges.shape[0]
    info = plsc.get_sparse_core_info()
    nw = info.num_cores * info.num_subcores
    per_tile = e_total // nw
    n_chunks = per_tile // _SC_CHUNK
    mesh = plsc.VectorSubcoreMesh(core_axis_name="c", subcore_axis_name="s")

    @functools.partial(
        pl.kernel,
        mesh=mesh,
        compiler_params=pltpu.CompilerParams(needs_layout_passes=False),
        out_type=[jax.ShapeDtypeStruct((e_total,), jnp.float32)] * 3,
        scratch_types=[
            pltpu.VMEM((npts,), jnp.float32),
            pltpu.VMEM((npts,), jnp.float32),
            pltpu.VMEM((npts,), jnp.float32),
            pltpu.VMEM((_SC_CHUNK,), jnp.int32),
            pltpu.VMEM((_SC_CHUNK,), jnp.float32),
            pltpu.VMEM((_SC_CHUNK,), jnp.float32),
            pltpu.VMEM((_SC_CHUNK,), jnp.float32),
        ],
    )
    def k(px_hbm, py_hbm, pz_hbm, e_hbm, ox_hbm, oy_hbm, oz_hbm,
          pxv, pyv, pzv, idxv, bx, by, bz):
        wid = lax.axis_index("s") * info.num_cores + lax.axis_index("c")
        pltpu.sync_copy(px_hbm, pxv)
        pltpu.sync_copy(py_hbm, pyv)
        pltpu.sync_copy(pz_hbm, pzv)
        lane = lax.iota(jnp.int32, 16)

        def chunk(c, _):
            base = wid * per_tile + c * _SC_CHUNK
            pltpu.sync_copy(e_hbm.at[pl.ds(base, _SC_CHUNK)], idxv)

            def group(g, _):
                idx16 = idxv[pl.ds(g * 16, 16)]
                org16 = lax.shift_right_logical(base + g * 16 + lane, 5)
                bx[pl.ds(g * 16, 16)] = (
                    plsc.load_gather(pxv, [idx16])
                    - plsc.load_gather(pxv, [org16]))
                by[pl.ds(g * 16, 16)] = (
                    plsc.load_gather(pyv, [idx16])
                    - plsc.load_gather(pyv, [org16]))
                bz[pl.ds(g * 16, 16)] = (
                    plsc.load_gather(pzv, [idx16])
                    - plsc.load_gather(pzv, [org16]))
                return 0

            lax.fori_loop(0, _SC_CHUNK // 16, group, 0)
            pltpu.sync_copy(bx, ox_hbm.at[pl.ds(base, _SC_CHUNK)])
            pltpu.sync_copy(by, oy_hbm.at[pl.ds(base, _SC_CHUNK)])
            pltpu.sync_copy(bz, oz_hbm.at[pl.ds(base, _SC_CHUNK)])
            return 0

        lax.fori_loop(0, n_chunks, chunk, 0)

    return k(px, py, pz, edges)


def _shard_fn(pc_local):
    bl, n, _ = pc_local.shape
    edges_local = _topk_edges(pc_local)
    pcf = pc_local.reshape(bl * n, 3)
    fx, fy, fz = _edge_feats(pcf[:, 0], pcf[:, 1], pcf[:, 2], edges_local)
    feats = jnp.stack([fx, fy, fz], axis=-1)
    edges = edges_local + lax.axis_index("d") * (bl * n)
    return edges, feats


def kernel(pc):
    nd = 2 if len(jax.devices()) >= 2 else 1
    mesh = Mesh(np.array(jax.devices()[:nd]), ("d",))
    edges, feats = jax.shard_map(
        _shard_fn, mesh=mesh, in_specs=P("d"), out_specs=(P("d"), P("d")),
        check_vma=False,
    )(pc)
    return edges, feats


# speedup vs baseline: 25.5597x; 1.0291x over previous
import functools

import jax
import jax.numpy as jnp
import numpy as np
from jax import lax
from jax.experimental import pallas as pl
from jax.experimental.pallas import tpu as pltpu
from jax.experimental.pallas import tpu_sc as plsc
from jax.sharding import Mesh, PartitionSpec as P

_N = 4096
_K = 32
_R = 512
_SC_CHUNK = 4096


def _topk_body(pc_ref, pcrt_ref, out_ref, dist_ref, sq_ref):
    b = pl.program_id(0)
    i = pl.program_id(1)
    pc_all = pc_ref[0]
    prt = pcrt_ref[0]

    @pl.when(i == 0)
    def _():
        sq_ref[...] = jnp.sum(pc_all * pc_all, axis=1, keepdims=True)

    sq_j = sq_ref[...]
    sq_r = jnp.sum(prt * prt, axis=0, keepdims=True)
    dot = lax.dot_general(pc_all, prt, (((1,), (0,)), ((), ())),
                          preferred_element_type=jnp.float32)
    dist_ref[...] = (sq_j + sq_r) - 2.0 * dot

    iota_c = lax.broadcasted_iota(jnp.int32, (_N, _R), 0)
    sel_iota = lax.broadcasted_iota(jnp.int32, (_K, _R), 0)

    def step(t, carry):
        acc, idx_prev = carry
        d = jnp.where(iota_c == idx_prev, jnp.inf, dist_ref[...])
        dist_ref[...] = d
        m = jnp.min(d, axis=0, keepdims=True)
        iw = jnp.where(d == m, iota_c, _N)
        idx = jnp.min(iw, axis=0, keepdims=True)
        acc = jnp.where(sel_iota == t, jnp.broadcast_to(idx, (_K, _R)), acc)
        return acc, idx

    acc, _ = lax.fori_loop(
        0, _K, step,
        (jnp.zeros((_K, _R), jnp.int32), jnp.full((1, _R), -1, jnp.int32)))
    out_ref[0] = acc + b * _N


def _topk_edges(pc):
    B = pc.shape[0]
    pcT = jnp.swapaxes(pc, 1, 2)
    edges_t = pl.pallas_call(
        _topk_body,
        grid=(B, _N // _R),
        in_specs=[
            pl.BlockSpec((1, _N, 3), lambda b, i: (b, 0, 0)),
            pl.BlockSpec((1, 3, _R), lambda b, i: (b, 0, i)),
        ],
        out_specs=pl.BlockSpec((1, _K, _R), lambda b, i: (b, 0, i)),
        out_shape=jax.ShapeDtypeStruct((B, _K, _N), jnp.int32),
        scratch_shapes=[
            pltpu.VMEM((_N, _R), jnp.float32),
            pltpu.VMEM((_N, 1), jnp.float32),
        ],
    )(pc, pcT)
    return jnp.swapaxes(edges_t, 1, 2).reshape(-1)


def _edge_feats(px, py, pz, edges):
    npts = px.shape[0]
    e_total = edges.shape[0]
    info = plsc.get_sparse_core_info()
    nw = info.num_cores * info.num_subcores
    per_tile = e_total // nw
    n_chunks = per_tile // _SC_CHUNK
    mesh = plsc.VectorSubcoreMesh(core_axis_name="c", subcore_axis_name="s")

    @functools.partial(
        pl.kernel,
        mesh=mesh,
        compiler_params=pltpu.CompilerParams(needs_layout_passes=False),
        out_type=[jax.ShapeDtypeStruct((e_total,), jnp.float32)] * 3,
        scratch_types=[
            pltpu.VMEM((npts,), jnp.float32),
            pltpu.VMEM((npts,), jnp.float32),
            pltpu.VMEM((npts,), jnp.float32),
            pltpu.VMEM((_SC_CHUNK,), jnp.int32),
            pltpu.VMEM((_SC_CHUNK,), jnp.float32),
            pltpu.VMEM((_SC_CHUNK,), jnp.float32),
            pltpu.VMEM((_SC_CHUNK,), jnp.float32),
        ],
    )
    def k(px_hbm, py_hbm, pz_hbm, e_hbm, ox_hbm, oy_hbm, oz_hbm,
          pxv, pyv, pzv, idxv, bx, by, bz):
        wid = lax.axis_index("s") * info.num_cores + lax.axis_index("c")
        pltpu.sync_copy(px_hbm, pxv)
        pltpu.sync_copy(py_hbm, pyv)
        pltpu.sync_copy(pz_hbm, pzv)
        lane = lax.iota(jnp.int32, 16)

        def chunk(c, _):
            base = wid * per_tile + c * _SC_CHUNK
            pltpu.sync_copy(e_hbm.at[pl.ds(base, _SC_CHUNK)], idxv)

            def group(g, _):
                idx16 = idxv[pl.ds(g * 16, 16)]
                org16 = lax.shift_right_logical(base + g * 16 + lane, 5)
                bx[pl.ds(g * 16, 16)] = (
                    plsc.load_gather(pxv, [idx16])
                    - plsc.load_gather(pxv, [org16]))
                by[pl.ds(g * 16, 16)] = (
                    plsc.load_gather(pyv, [idx16])
                    - plsc.load_gather(pyv, [org16]))
                bz[pl.ds(g * 16, 16)] = (
                    plsc.load_gather(pzv, [idx16])
                    - plsc.load_gather(pzv, [org16]))
                return 0

            lax.fori_loop(0, _SC_CHUNK // 16, group, 0)
            pltpu.sync_copy(bx, ox_hbm.at[pl.ds(base, _SC_CHUNK)])
            pltpu.sync_copy(by, oy_hbm.at[pl.ds(base, _SC_CHUNK)])
            pltpu.sync_copy(bz, oz_hbm.at[pl.ds(base, _SC_CHUNK)])
            return 0

        lax.fori_loop(0, n_chunks, chunk, 0)

    return k(px, py, pz, edges)


def _shard_fn(pc_local):
    bl, n, _ = pc_local.shape
    edges_local = _topk_edges(pc_local)
    pcf = pc_local.reshape(bl * n, 3)
    fx, fy, fz = _edge_feats(pcf[:, 0], pcf[:, 1], pcf[:, 2], edges_local)
    feats = jnp.stack([fx, fy, fz], axis=-1)
    edges = edges_local + lax.axis_index("d") * (bl * n)
    return edges, feats


def kernel(pc):
    nd = 2 if len(jax.devices()) >= 2 else 1
    mesh = Mesh(np.array(jax.devices()[:nd]), ("d",))
    edges, feats = jax.shard_map(
        _shard_fn, mesh=mesh, in_specs=P("d"), out_specs=(P("d"), P("d")),
        check_vma=False,
    )(pc)
    return edges, feats
